# baseline (speedup 1.0000x reference)
import functools

import jax
import jax.numpy as jnp
from jax.experimental import pallas as pl
from jax.experimental.pallas import tpu as pltpu

_TOKEN_TILE = 1024


def _mlp_kernel(x_ref, w1_ref, b1_ref, w2_ref, b2_ref,
                logits_ref, probs_ref, acts_ref):
    x = x_ref[...]
    hidden = jnp.maximum(
        jnp.dot(x, w1_ref[...], preferred_element_type=jnp.float32)
        + b1_ref[...], 0.0)
    logits = (jnp.dot(hidden, w2_ref[...], preferred_element_type=jnp.float32)
              + b2_ref[...])
    probs = jax.nn.sigmoid(logits)
    logits_ref[...] = logits
    probs_ref[...] = probs
    acts_ref[...] = (probs > 0.5).astype(jnp.float32)


@jax.jit
def kernel(neural_repr, W1, b1, W2, b2):
    tokens, embed = neural_repr.shape
    hidden = W1.shape[1]
    num_symbols = W2.shape[1]
    tile = min(_TOKEN_TILE, tokens)
    grid = (tokens // tile,)

    out_shape = [
        jax.ShapeDtypeStruct((tokens, num_symbols), jnp.float32)
        for _ in range(3)
    ]
    out_spec = pl.BlockSpec((tile, num_symbols), lambda i: (i, 0))

    logits, probs, acts = pl.pallas_call(
        _mlp_kernel,
        grid=grid,
        in_specs=[
            pl.BlockSpec((tile, embed), lambda i: (i, 0)),
            pl.BlockSpec((embed, hidden), lambda i: (0, 0)),
            pl.BlockSpec((1, hidden), lambda i: (0, 0)),
            pl.BlockSpec((hidden, num_symbols), lambda i: (0, 0)),
            pl.BlockSpec((1, num_symbols), lambda i: (0, 0)),
        ],
        out_specs=[out_spec, out_spec, out_spec],
        out_shape=out_shape,
        compiler_params=pltpu.CompilerParams(
            dimension_semantics=("arbitrary",),
        ),
    )(neural_repr, W1, b1.reshape(1, hidden), W2, b2.reshape(1, num_symbols))
    return (logits, probs, acts)


# speedup vs baseline: 1.0148x; 1.0148x over previous
import functools

import jax
import jax.numpy as jnp
from jax.experimental import pallas as pl
from jax.experimental.pallas import tpu as pltpu

_TOKEN_TILE = 1024


def _mlp_kernel(x_ref, w1_ref, b1_ref, w2_ref, b2_ref,
                logits_ref, probs_ref, acts_ref):
    x = x_ref[...]
    hidden = jnp.maximum(
        jnp.dot(x, w1_ref[...], preferred_element_type=jnp.float32)
        + b1_ref[...], 0.0)
    logits = (jnp.dot(hidden, w2_ref[...], preferred_element_type=jnp.float32)
              + b2_ref[...])
    probs = 0.5 * jnp.tanh(0.5 * logits) + 0.5
    logits_ref[...] = logits
    probs_ref[...] = probs
    acts_ref[...] = (logits > 0.0).astype(jnp.float32)


@jax.jit
def kernel(neural_repr, W1, b1, W2, b2):
    tokens, embed = neural_repr.shape
    hidden = W1.shape[1]
    num_symbols = W2.shape[1]
    tile = min(_TOKEN_TILE, tokens)
    grid = (tokens // tile,)

    out_shape = [
        jax.ShapeDtypeStruct((tokens, num_symbols), jnp.float32)
        for _ in range(3)
    ]
    out_spec = pl.BlockSpec((tile, num_symbols), lambda i: (i, 0))

    logits, probs, acts = pl.pallas_call(
        _mlp_kernel,
        grid=grid,
        in_specs=[
            pl.BlockSpec((tile, embed), lambda i: (i, 0)),
            pl.BlockSpec((embed, hidden), lambda i: (0, 0)),
            pl.BlockSpec((1, hidden), lambda i: (0, 0)),
            pl.BlockSpec((hidden, num_symbols), lambda i: (0, 0)),
            pl.BlockSpec((1, num_symbols), lambda i: (0, 0)),
        ],
        out_specs=[out_spec, out_spec, out_spec],
        out_shape=out_shape,
        compiler_params=pltpu.CompilerParams(
            dimension_semantics=("arbitrary",),
        ),
    )(neural_repr, W1, b1.reshape(1, hidden), W2, b2.reshape(1, num_symbols))
    return (logits, probs, acts)
